# baseline (speedup 1.0000x reference)
import dataclasses
import functools

import jax
import jax.numpy as jnp
from jax import lax
from jax.experimental import pallas as pl
from jax.experimental.pallas import tpu as pltpu
from jax.experimental.pallas import tpu_sc as plsc

N_RAW = 10000
N_PAD = 10240
E_RAW = 160000
E_PAD = 163840
D = 256
DH = 128
D_OUT = 64
B_G = 100
B_PAD = 128
ROW_BLK = 512
N_BLKS = N_PAD // ROW_BLK


def _mm_body(x_ref, w_ref, o_ref):
    o_ref[...] = jnp.dot(x_ref[...], w_ref[...],
                         preferred_element_type=jnp.float32)


def _tc_matmul(x, w):
    return pl.pallas_call(
        _mm_body,
        grid=(N_BLKS,),
        in_specs=[
            pl.BlockSpec((ROW_BLK, D), lambda i: (i, 0)),
            pl.BlockSpec((D, D), lambda i: (0, 0)),
        ],
        out_specs=pl.BlockSpec((ROW_BLK, D), lambda i: (i, 0)),
        out_shape=jax.ShapeDtypeStruct((N_PAD, D), jnp.float32),
    )(x, w)


def _scale_body(m_ref, deg_ref, p_ref, norm_ref):
    deg = jnp.sum(deg_ref[...], axis=0)
    norm = lax.rsqrt(jnp.maximum(deg, 1.0))[:, None]
    p = m_ref[...] * norm
    p_ref[0] = p[:, :DH]
    p_ref[1] = p[:, DH:]
    norm_ref[...] = norm


def _tc_scale_split(m, deg_parts):
    return pl.pallas_call(
        _scale_body,
        grid=(N_BLKS,),
        in_specs=[
            pl.BlockSpec((ROW_BLK, D), lambda i: (i, 0)),
            pl.BlockSpec((N_WORKER, ROW_BLK), lambda i: (0, i)),
        ],
        out_specs=[
            pl.BlockSpec((2, ROW_BLK, DH), lambda i: (0, i, 0)),
            pl.BlockSpec((ROW_BLK, 1), lambda i: (i, 0)),
        ],
        out_shape=[
            jax.ShapeDtypeStruct((2, N_PAD, DH), jnp.float32),
            jax.ShapeDtypeStruct((N_PAD, 1), jnp.float32),
        ],
    )(m, deg_parts)


def _layer2_body(s_ref, norm_ref, b_ref, w_ref, p_ref):
    norm = norm_ref[...]
    x = jnp.concatenate([s_ref[0], s_ref[1]], axis=1)
    h = jax.nn.relu(x * norm + b_ref[...]) * norm
    p = jnp.dot(h, w_ref[...], preferred_element_type=jnp.float32)
    p_ref[0] = p[:, :DH]
    p_ref[1] = p[:, DH:]


def _tc_layer2(s_parts, norm, b1, w2):
    return pl.pallas_call(
        _layer2_body,
        grid=(N_BLKS,),
        in_specs=[
            pl.BlockSpec((2, ROW_BLK, DH), lambda i: (0, i, 0)),
            pl.BlockSpec((ROW_BLK, 1), lambda i: (i, 0)),
            pl.BlockSpec((1, D), lambda i: (0, 0)),
            pl.BlockSpec((D, D), lambda i: (0, 0)),
        ],
        out_specs=pl.BlockSpec((2, ROW_BLK, DH), lambda i: (0, i, 0)),
        out_shape=jax.ShapeDtypeStruct((2, N_PAD, DH), jnp.float32),
    )(s_parts, norm, b1, w2)


def _readout_body(fetch_ref, g_ref, norm_ref, b_ref, wlt_ref, bl_ref,
                  o_ref, gn):
    def body(i, _):
        idx = fetch_ref[i]
        gn[pl.ds(i, 1), :] = norm_ref[pl.ds(idx, 1), :]
        return 0

    lax.fori_loop(0, B_PAD, body, 0)
    x = jnp.concatenate([g_ref[0], g_ref[1]], axis=1)
    h = jax.nn.relu(x * gn[...] + b_ref[...])
    o_ref[...] = jnp.dot(h, wlt_ref[...],
                         preferred_element_type=jnp.float32) + bl_ref[...]


def _tc_readout(fetch_idx, gath, norm, b2, wl_t, bl):
    return pl.pallas_call(
        _readout_body,
        in_specs=[
            pl.BlockSpec(memory_space=pltpu.SMEM),
            pl.BlockSpec((2, B_PAD, DH), lambda: (0, 0, 0)),
            pl.BlockSpec((N_PAD, 1), lambda: (0, 0)),
            pl.BlockSpec((1, D), lambda: (0, 0)),
            pl.BlockSpec((D, D_OUT), lambda: (0, 0)),
            pl.BlockSpec((1, D_OUT), lambda: (0, 0)),
        ],
        out_specs=pl.BlockSpec((B_PAD, D_OUT), lambda: (0, 0)),
        out_shape=jax.ShapeDtypeStruct((B_PAD, D_OUT), jnp.float32),
        scratch_shapes=[
            pltpu.VMEM((B_PAD, 1), jnp.float32),
        ],
    )(fetch_idx, gath, norm, b2, wl_t, bl)


_SC_MESH = plsc.VectorSubcoreMesh(core_axis_name="c", subcore_axis_name="s")
_SC_CP = pltpu.CompilerParams()
if "needs_layout_passes" in pltpu.CompilerParams.__dataclass_fields__:
    _SC_CP = dataclasses.replace(_SC_CP, needs_layout_passes=False)
N_TEC = 16
N_WORKER = 32
E_PER_TEC = E_PAD // N_TEC
CHUNK = 128
N_CHUNKS = E_PER_TEC // CHUNK
N_PAIRS = N_CHUNKS // 2
ROWS_PER_TEC = N_PAD // N_TEC
E_PER_W = E_PAD // N_WORKER
LANES = 16
PHASES = 2
CPP = N_CHUNKS // PHASES


@functools.partial(
    pl.kernel,
    mesh=_SC_MESH,
    out_type=jax.ShapeDtypeStruct((N_WORKER, N_PAD), jnp.float32),
    scratch_types=[
        pltpu.VMEM((E_PER_W,), jnp.int32),
        pltpu.VMEM((N_PAD,), jnp.float32),
    ],
    compiler_params=_SC_CP,
)
def _sc_degree(dst_hbm, out_hbm, idx_v, hist_v):
    c = lax.axis_index("c")
    s = lax.axis_index("s")
    wid = s * 2 + c
    zeros = jnp.zeros((LANES,), jnp.float32)
    ones = jnp.full((LANES,), 1.0, jnp.float32)

    @pl.loop(0, N_PAD // LANES)
    def _(i):
        hist_v[pl.ds(i * LANES, LANES)] = zeros

    pltpu.sync_copy(dst_hbm.at[pl.ds(wid * E_PER_W, E_PER_W)], idx_v)

    @pl.loop(0, E_PER_W // LANES)
    def _(i):
        idx = idx_v[pl.ds(i * LANES, LANES)]
        plsc.addupdate_scatter(hist_v, [idx], ones)

    pltpu.sync_copy(hist_v, out_hbm.at[wid])


@functools.partial(
    pl.kernel,
    mesh=_SC_MESH,
    out_type=[
        jax.ShapeDtypeStruct((2 * N_PAD, DH), jnp.float32),
        jax.ShapeDtypeStruct((2, B_PAD, DH), jnp.float32),
    ],
    scratch_types=[
        pltpu.VMEM((CPP, CHUNK), jnp.int32),
        pltpu.VMEM((CPP, CHUNK), jnp.int32),
        pltpu.VMEM((CHUNK, DH), jnp.float32),
        pltpu.VMEM((CHUNK, DH), jnp.float32),
        pltpu.VMEM_SHARED((N_PAD, DH), jnp.float32),
        pltpu.SemaphoreType.DMA,
        pltpu.SemaphoreType.DMA,
    ],
    compiler_params=_SC_CP,
)
def _sc_scatter(p_hbm, srcp_hbm, dstp_hbm, fetch_hbm, out_hbm, gath_hbm,
                src_v, dst_v, rows0, rows1, acc_sh, sr0, sr1):
    c = lax.axis_index("c")
    s = lax.axis_index("s")
    zeros = jnp.zeros((LANES,), jnp.float32)

    @pl.loop(0, CHUNK)
    def _(r):
        @pl.loop(0, DH // LANES)
        def _(k):
            rows0[r, pl.ds(k * LANES, LANES)] = zeros

    @pl.loop(0, ROWS_PER_TEC // CHUNK)
    def _(b):
        pltpu.sync_copy(rows0,
                        acc_sh.at[pl.ds(s * ROWS_PER_TEC + b * CHUNK, CHUNK)])

    plsc.subcore_barrier()

    def gat(j, rows, sem):
        return pltpu.make_async_copy(p_hbm.at[src_v.at[j]], rows, sem)

    def sca(j, rows):
        pltpu.sync_copy(rows, acc_sh.at[dst_v.at[j]], add=True)

    for p in range(PHASES):
        pltpu.sync_copy(srcp_hbm.at[c, s, p], src_v)
        pltpu.sync_copy(dstp_hbm.at[s, p], dst_v)
        gat(0, rows0, sr0).start()

        @pl.loop(0, CPP // 2 - 1)
        def _(k):
            j0 = 2 * k
            gat(j0 + 1, rows1, sr1).start()
            gat(j0, rows0, sr0).wait()
            sca(j0, rows0)
            gat(j0 + 2, rows0, sr0).start()
            gat(j0 + 1, rows1, sr1).wait()
            sca(j0 + 1, rows1)

        gat(CPP - 1, rows1, sr1).start()
        gat(CPP - 2, rows0, sr0).wait()
        sca(CPP - 2, rows0)
        gat(CPP - 1, rows1, sr1).wait()
        sca(CPP - 1, rows1)

    plsc.subcore_barrier()
    pltpu.sync_copy(
        acc_sh.at[pl.ds(s * ROWS_PER_TEC, ROWS_PER_TEC)],
        out_hbm.at[pl.ds(c * N_PAD + s * ROWS_PER_TEC, ROWS_PER_TEC)])
    plsc.subcore_barrier()

    @pl.when(s == 0)
    def _():
        pltpu.sync_copy(fetch_hbm, src_v.at[0])

        @pl.loop(0, B_PAD // LANES)
        def _(i):
            v = src_v[0, pl.ds(i * LANES, LANES)]
            src_v[0, pl.ds(i * LANES, LANES)] = v + c * N_PAD

        pltpu.async_copy(out_hbm.at[src_v.at[0]], rows0, sr0).wait()
        pltpu.sync_copy(rows0, gath_hbm.at[c])


def _deg_parts(dst_pad_flat):
    return _sc_degree(dst_pad_flat)


def _scatter_parts(p_parts, srcp, dstp, fetch_idx):
    p_flat = p_parts.reshape(2 * N_PAD, DH)
    s_flat, gath = _sc_scatter(p_flat, srcp, dstp, fetch_idx)
    return s_flat.reshape(2, N_PAD, DH), gath


def kernel(features, edge_index, to_fetch, W1, b1, W2, b2, Wl, bl):
    src = edge_index[0].astype(jnp.int32)
    dst = edge_index[1].astype(jnp.int32)

    junk = N_RAW + jnp.arange(E_PAD - E_RAW, dtype=jnp.int32) % (N_PAD - N_RAW)
    src_pad = jnp.concatenate([src, junk])
    dst_pad = jnp.concatenate([dst, junk])
    srcp = jnp.stack([src_pad, src_pad + N_PAD]).reshape(
        2, N_TEC, PHASES, CPP, CHUNK)
    dstp = dst_pad.reshape(N_TEC, PHASES, CPP, CHUNK)

    feats = jnp.zeros((N_PAD, D), jnp.float32).at[:N_RAW].set(
        features.astype(jnp.float32))
    fetch_idx = jnp.zeros((B_PAD,), jnp.int32).at[:B_G].set(
        to_fetch.astype(jnp.int32)
        + jnp.arange(B_G, dtype=jnp.int32) * (N_RAW // B_G))

    deg_parts = _deg_parts(dst_pad)

    m1 = _tc_matmul(feats, W1)
    p1, norm = _tc_scale_split(m1, deg_parts)
    s1, _ = _scatter_parts(p1, srcp, dstp, fetch_idx)
    p2 = _tc_layer2(s1, norm, b1.reshape(1, D), W2)
    _, gath2 = _scatter_parts(p2, srcp, dstp, fetch_idx)
    out = _tc_readout(fetch_idx, gath2, norm, b2.reshape(1, D),
                      Wl.T, bl.reshape(1, D_OUT))
    h = out[:B_G]
    return (h, h)


# speedup vs baseline: 1.0154x; 1.0154x over previous
import dataclasses
import functools

import jax
import jax.numpy as jnp
from jax import lax
from jax.experimental import pallas as pl
from jax.experimental.pallas import tpu as pltpu
from jax.experimental.pallas import tpu_sc as plsc

N_RAW = 10000
N_PAD = 10240
E_RAW = 160000
E_PAD = 163840
D = 256
DH = 128
D_OUT = 64
B_G = 100
B_PAD = 128
ROW_BLK = 512
N_BLKS = N_PAD // ROW_BLK


def _mm_body(x_ref, w_ref, o_ref):
    o_ref[...] = jnp.dot(x_ref[...], w_ref[...],
                         preferred_element_type=jnp.float32)


def _tc_matmul(x, w):
    return pl.pallas_call(
        _mm_body,
        grid=(N_BLKS,),
        in_specs=[
            pl.BlockSpec((ROW_BLK, D), lambda i: (i, 0)),
            pl.BlockSpec((D, D), lambda i: (0, 0)),
        ],
        out_specs=pl.BlockSpec((ROW_BLK, D), lambda i: (i, 0)),
        out_shape=jax.ShapeDtypeStruct((N_PAD, D), jnp.float32),
    )(x, w)


def _scale_body(m_ref, deg_ref, p_ref, norm_ref):
    deg = jnp.sum(deg_ref[...], axis=0)
    norm = lax.rsqrt(jnp.maximum(deg, 1.0))[:, None]
    p = m_ref[...] * norm
    p_ref[0] = p[:, :DH]
    p_ref[1] = p[:, DH:]
    norm_ref[...] = norm


def _tc_scale_split(m, deg_parts):
    return pl.pallas_call(
        _scale_body,
        grid=(N_BLKS,),
        in_specs=[
            pl.BlockSpec((ROW_BLK, D), lambda i: (i, 0)),
            pl.BlockSpec((N_WORKER, ROW_BLK), lambda i: (0, i)),
        ],
        out_specs=[
            pl.BlockSpec((2, ROW_BLK, DH), lambda i: (0, i, 0)),
            pl.BlockSpec((ROW_BLK, 1), lambda i: (i, 0)),
        ],
        out_shape=[
            jax.ShapeDtypeStruct((2, N_PAD, DH), jnp.float32),
            jax.ShapeDtypeStruct((N_PAD, 1), jnp.float32),
        ],
    )(m, deg_parts)


def _layer2_body(s_ref, norm_ref, b_ref, w_ref, p_ref):
    norm = norm_ref[...]
    x = jnp.concatenate([s_ref[0], s_ref[1]], axis=1)
    h = jax.nn.relu(x * norm + b_ref[...]) * norm
    p = jnp.dot(h, w_ref[...], preferred_element_type=jnp.float32)
    p_ref[0] = p[:, :DH]
    p_ref[1] = p[:, DH:]


def _tc_layer2(s_parts, norm, b1, w2):
    return pl.pallas_call(
        _layer2_body,
        grid=(N_BLKS,),
        in_specs=[
            pl.BlockSpec((2, ROW_BLK, DH), lambda i: (0, i, 0)),
            pl.BlockSpec((ROW_BLK, 1), lambda i: (i, 0)),
            pl.BlockSpec((1, D), lambda i: (0, 0)),
            pl.BlockSpec((D, D), lambda i: (0, 0)),
        ],
        out_specs=pl.BlockSpec((2, ROW_BLK, DH), lambda i: (0, i, 0)),
        out_shape=jax.ShapeDtypeStruct((2, N_PAD, DH), jnp.float32),
    )(s_parts, norm, b1, w2)


def _readout_body(fetch_ref, g_ref, norm_ref, b_ref, wlt_ref, bl_ref,
                  o_ref, gn):
    def body(i, _):
        idx = fetch_ref[i]
        gn[pl.ds(i, 1), :] = norm_ref[pl.ds(idx, 1), :]
        return 0

    lax.fori_loop(0, B_PAD, body, 0)
    x = jnp.concatenate([g_ref[0], g_ref[1]], axis=1)
    h = jax.nn.relu(x * gn[...] + b_ref[...])
    o_ref[...] = jnp.dot(h, wlt_ref[...],
                         preferred_element_type=jnp.float32) + bl_ref[...]


def _tc_readout(fetch_idx, gath, norm, b2, wl_t, bl):
    return pl.pallas_call(
        _readout_body,
        in_specs=[
            pl.BlockSpec(memory_space=pltpu.SMEM),
            pl.BlockSpec((2, B_PAD, DH), lambda: (0, 0, 0)),
            pl.BlockSpec((N_PAD, 1), lambda: (0, 0)),
            pl.BlockSpec((1, D), lambda: (0, 0)),
            pl.BlockSpec((D, D_OUT), lambda: (0, 0)),
            pl.BlockSpec((1, D_OUT), lambda: (0, 0)),
        ],
        out_specs=pl.BlockSpec((B_PAD, D_OUT), lambda: (0, 0)),
        out_shape=jax.ShapeDtypeStruct((B_PAD, D_OUT), jnp.float32),
        scratch_shapes=[
            pltpu.VMEM((B_PAD, 1), jnp.float32),
        ],
    )(fetch_idx, gath, norm, b2, wl_t, bl)


_SC_MESH = plsc.VectorSubcoreMesh(core_axis_name="c", subcore_axis_name="s")
_SC_CP = pltpu.CompilerParams()
if "needs_layout_passes" in pltpu.CompilerParams.__dataclass_fields__:
    _SC_CP = dataclasses.replace(_SC_CP, needs_layout_passes=False)
N_TEC = 16
N_WORKER = 32
E_PER_TEC = E_PAD // N_TEC
CHUNK = 128
N_CHUNKS = E_PER_TEC // CHUNK
N_PAIRS = N_CHUNKS // 2
ROWS_PER_TEC = N_PAD // N_TEC
E_PER_W = E_PAD // N_WORKER
LANES = 16
PHASES = 2
CPP = N_CHUNKS // PHASES


@functools.partial(
    pl.kernel,
    mesh=_SC_MESH,
    out_type=jax.ShapeDtypeStruct((N_WORKER, N_PAD), jnp.float32),
    scratch_types=[
        pltpu.VMEM((E_PER_W,), jnp.int32),
        pltpu.VMEM((N_PAD,), jnp.float32),
    ],
    compiler_params=_SC_CP,
)
def _sc_degree(dst_hbm, out_hbm, idx_v, hist_v):
    c = lax.axis_index("c")
    s = lax.axis_index("s")
    wid = s * 2 + c
    zeros = jnp.zeros((LANES,), jnp.float32)
    ones = jnp.full((LANES,), 1.0, jnp.float32)

    @pl.loop(0, N_PAD // LANES)
    def _(i):
        hist_v[pl.ds(i * LANES, LANES)] = zeros

    pltpu.sync_copy(dst_hbm.at[pl.ds(wid * E_PER_W, E_PER_W)], idx_v)

    @pl.loop(0, E_PER_W // LANES)
    def _(i):
        idx = idx_v[pl.ds(i * LANES, LANES)]
        plsc.addupdate_scatter(hist_v, [idx], ones)

    pltpu.sync_copy(hist_v, out_hbm.at[wid])


def _make_sc_scatter(with_gather):
    if with_gather:
        out_type = [
            jax.ShapeDtypeStruct((2 * N_PAD, DH), jnp.float32),
            jax.ShapeDtypeStruct((2, B_PAD, DH), jnp.float32),
        ]
    else:
        out_type = jax.ShapeDtypeStruct((2 * N_PAD, DH), jnp.float32)

    @functools.partial(
        pl.kernel,
        mesh=_SC_MESH,
        out_type=out_type,
        scratch_types=[
            pltpu.VMEM((CPP, CHUNK), jnp.int32),
            pltpu.VMEM((CPP, CHUNK), jnp.int32),
            pltpu.VMEM((CHUNK, DH), jnp.float32),
            pltpu.VMEM((CHUNK, DH), jnp.float32),
            pltpu.VMEM_SHARED((N_PAD, DH), jnp.float32),
            pltpu.SemaphoreType.DMA,
            pltpu.SemaphoreType.DMA,
        ],
        compiler_params=_SC_CP,
    )
    def _sc_scatter(p_hbm, srcp_hbm, dstp_hbm, fetch_hbm, *refs):
        if with_gather:
            (out_hbm, gath_hbm,
             src_v, dst_v, rows0, rows1, acc_sh, sr0, sr1) = refs
        else:
            (out_hbm,
             src_v, dst_v, rows0, rows1, acc_sh, sr0, sr1) = refs
        c = lax.axis_index("c")
        s = lax.axis_index("s")
        zeros = jnp.zeros((LANES,), jnp.float32)

        @pl.loop(0, CHUNK)
        def _(r):
            @pl.loop(0, DH // LANES)
            def _(k):
                rows0[r, pl.ds(k * LANES, LANES)] = zeros

        @pl.loop(0, ROWS_PER_TEC // CHUNK)
        def _(b):
            pltpu.sync_copy(
                rows0,
                acc_sh.at[pl.ds(s * ROWS_PER_TEC + b * CHUNK, CHUNK)])

        plsc.subcore_barrier()

        def gat(j, rows, sem):
            return pltpu.make_async_copy(p_hbm.at[src_v.at[j]], rows, sem)

        def sca(j, rows):
            pltpu.sync_copy(rows, acc_sh.at[dst_v.at[j]], add=True)

        for p in range(PHASES):
            pltpu.sync_copy(srcp_hbm.at[c, s, p], src_v)
            pltpu.sync_copy(dstp_hbm.at[s, p], dst_v)
            gat(0, rows0, sr0).start()

            @pl.loop(0, CPP // 2 - 1)
            def _(k):
                j0 = 2 * k
                gat(j0 + 1, rows1, sr1).start()
                gat(j0, rows0, sr0).wait()
                sca(j0, rows0)
                gat(j0 + 2, rows0, sr0).start()
                gat(j0 + 1, rows1, sr1).wait()
                sca(j0 + 1, rows1)

            gat(CPP - 1, rows1, sr1).start()
            gat(CPP - 2, rows0, sr0).wait()
            sca(CPP - 2, rows0)
            gat(CPP - 1, rows1, sr1).wait()
            sca(CPP - 1, rows1)

        plsc.subcore_barrier()
        pltpu.sync_copy(
            acc_sh.at[pl.ds(s * ROWS_PER_TEC, ROWS_PER_TEC)],
            out_hbm.at[pl.ds(c * N_PAD + s * ROWS_PER_TEC, ROWS_PER_TEC)])

        if with_gather:
            plsc.subcore_barrier()

            @pl.when(s == 0)
            def _():
                pltpu.sync_copy(fetch_hbm, src_v.at[0])

                @pl.loop(0, B_PAD // LANES)
                def _(i):
                    v = src_v[0, pl.ds(i * LANES, LANES)]
                    src_v[0, pl.ds(i * LANES, LANES)] = v + c * N_PAD

                pltpu.async_copy(out_hbm.at[src_v.at[0]], rows0, sr0).wait()
                pltpu.sync_copy(rows0, gath_hbm.at[c])

    return _sc_scatter


_sc_scatter_plain = _make_sc_scatter(False)
_sc_scatter_gath = _make_sc_scatter(True)


def _deg_parts(dst_pad_flat):
    return _sc_degree(dst_pad_flat)


def _scatter_parts(p_parts, srcp, dstp, fetch_idx):
    p_flat = p_parts.reshape(2 * N_PAD, DH)
    s_flat = _sc_scatter_plain(p_flat, srcp, dstp, fetch_idx)
    return s_flat.reshape(2, N_PAD, DH)


def _scatter_parts_gath(p_parts, srcp, dstp, fetch_idx):
    p_flat = p_parts.reshape(2 * N_PAD, DH)
    _, gath = _sc_scatter_gath(p_flat, srcp, dstp, fetch_idx)
    return gath


def kernel(features, edge_index, to_fetch, W1, b1, W2, b2, Wl, bl):
    src = edge_index[0].astype(jnp.int32)
    dst = edge_index[1].astype(jnp.int32)

    junk = N_RAW + jnp.arange(E_PAD - E_RAW, dtype=jnp.int32) % (N_PAD - N_RAW)
    src_pad = jnp.concatenate([src, junk])
    dst_pad = jnp.concatenate([dst, junk])
    srcp = jnp.stack([src_pad, src_pad + N_PAD]).reshape(
        2, N_TEC, PHASES, CPP, CHUNK)
    dstp = dst_pad.reshape(N_TEC, PHASES, CPP, CHUNK)

    feats = jnp.zeros((N_PAD, D), jnp.float32).at[:N_RAW].set(
        features.astype(jnp.float32))
    fetch_idx = jnp.zeros((B_PAD,), jnp.int32).at[:B_G].set(
        to_fetch.astype(jnp.int32)
        + jnp.arange(B_G, dtype=jnp.int32) * (N_RAW // B_G))

    deg_parts = _deg_parts(dst_pad)

    m1 = _tc_matmul(feats, W1)
    p1, norm = _tc_scale_split(m1, deg_parts)
    s1 = _scatter_parts(p1, srcp, dstp, fetch_idx)
    p2 = _tc_layer2(s1, norm, b1.reshape(1, D), W2)
    gath2 = _scatter_parts_gath(p2, srcp, dstp, fetch_idx)
    out = _tc_readout(fetch_idx, gath2, norm, b2.reshape(1, D),
                      Wl.T, bl.reshape(1, D_OUT))
    h = out[:B_G]
    return (h, h)


# speedup vs baseline: 1.0473x; 1.0315x over previous
import dataclasses
import functools

import jax
import jax.numpy as jnp
from jax import lax
from jax.experimental import pallas as pl
from jax.experimental.pallas import tpu as pltpu
from jax.experimental.pallas import tpu_sc as plsc

N_RAW = 10000
N_PAD = 10240
E_RAW = 160000
E_PAD = 163840
D = 256
DH = 128
D_OUT = 64
B_G = 100
B_PAD = 128
ROW_BLK = 512
N_BLKS = N_PAD // ROW_BLK


def _mm_body(x_ref, w_ref, o_ref):
    o_ref[...] = jnp.dot(x_ref[...], w_ref[...],
                         preferred_element_type=jnp.float32)


def _tc_matmul(x, w):
    return pl.pallas_call(
        _mm_body,
        grid=(N_BLKS,),
        in_specs=[
            pl.BlockSpec((ROW_BLK, D), lambda i: (i, 0)),
            pl.BlockSpec((D, D), lambda i: (0, 0)),
        ],
        out_specs=pl.BlockSpec((ROW_BLK, D), lambda i: (i, 0)),
        out_shape=jax.ShapeDtypeStruct((N_PAD, D), jnp.float32),
    )(x, w)


def _scale_body(m_ref, deg_ref, p_ref, norm_ref):
    deg = jnp.sum(deg_ref[...], axis=0)
    norm = lax.rsqrt(jnp.maximum(deg, 1.0))[:, None]
    p = m_ref[...] * norm
    p_ref[0] = p[:, :DH]
    p_ref[1] = p[:, DH:]
    norm_ref[...] = norm


def _tc_scale_split(m, deg_parts):
    return pl.pallas_call(
        _scale_body,
        grid=(N_BLKS,),
        in_specs=[
            pl.BlockSpec((ROW_BLK, D), lambda i: (i, 0)),
            pl.BlockSpec((N_WORKER, ROW_BLK), lambda i: (0, i)),
        ],
        out_specs=[
            pl.BlockSpec((2, ROW_BLK, DH), lambda i: (0, i, 0)),
            pl.BlockSpec((ROW_BLK, 1), lambda i: (i, 0)),
        ],
        out_shape=[
            jax.ShapeDtypeStruct((2, N_PAD, DH), jnp.float32),
            jax.ShapeDtypeStruct((N_PAD, 1), jnp.float32),
        ],
    )(m, deg_parts)


def _layer2_body(s_ref, norm_ref, b_ref, w_ref, p_ref):
    norm = norm_ref[...]
    x = jnp.concatenate([s_ref[0], s_ref[1]], axis=1)
    h = jax.nn.relu(x * norm + b_ref[...]) * norm
    p = jnp.dot(h, w_ref[...], preferred_element_type=jnp.float32)
    p_ref[0] = p[:, :DH]
    p_ref[1] = p[:, DH:]


def _tc_layer2(s_parts, norm, b1, w2):
    return pl.pallas_call(
        _layer2_body,
        grid=(N_BLKS,),
        in_specs=[
            pl.BlockSpec((2, ROW_BLK, DH), lambda i: (0, i, 0)),
            pl.BlockSpec((ROW_BLK, 1), lambda i: (i, 0)),
            pl.BlockSpec((1, D), lambda i: (0, 0)),
            pl.BlockSpec((D, D), lambda i: (0, 0)),
        ],
        out_specs=pl.BlockSpec((2, ROW_BLK, DH), lambda i: (0, i, 0)),
        out_shape=jax.ShapeDtypeStruct((2, N_PAD, DH), jnp.float32),
    )(s_parts, norm, b1, w2)


def _readout_body(fetch_ref, g_ref, norm_ref, b_ref, wlt_ref, bl_ref,
                  o_ref, gn):
    def body(i, _):
        idx = fetch_ref[i]
        gn[pl.ds(i, 1), :] = norm_ref[pl.ds(idx, 1), :]
        return 0

    lax.fori_loop(0, B_PAD, body, 0)
    x = jnp.concatenate([g_ref[0], g_ref[1]], axis=1)
    h = jax.nn.relu(x * gn[...] + b_ref[...])
    o_ref[...] = jnp.dot(h, wlt_ref[...],
                         preferred_element_type=jnp.float32) + bl_ref[...]


def _tc_readout(fetch_idx, gath, norm, b2, wl_t, bl):
    return pl.pallas_call(
        _readout_body,
        in_specs=[
            pl.BlockSpec(memory_space=pltpu.SMEM),
            pl.BlockSpec((2, B_PAD, DH), lambda: (0, 0, 0)),
            pl.BlockSpec((N_PAD, 1), lambda: (0, 0)),
            pl.BlockSpec((1, D), lambda: (0, 0)),
            pl.BlockSpec((D, D_OUT), lambda: (0, 0)),
            pl.BlockSpec((1, D_OUT), lambda: (0, 0)),
        ],
        out_specs=pl.BlockSpec((B_PAD, D_OUT), lambda: (0, 0)),
        out_shape=jax.ShapeDtypeStruct((B_PAD, D_OUT), jnp.float32),
        scratch_shapes=[
            pltpu.VMEM((B_PAD, 1), jnp.float32),
        ],
    )(fetch_idx, gath, norm, b2, wl_t, bl)


_SC_MESH = plsc.VectorSubcoreMesh(core_axis_name="c", subcore_axis_name="s")
_SC_CP = pltpu.CompilerParams()
if "needs_layout_passes" in pltpu.CompilerParams.__dataclass_fields__:
    _SC_CP = dataclasses.replace(_SC_CP, needs_layout_passes=False)
N_TEC = 16
N_WORKER = 32
E_PER_TEC = E_PAD // N_TEC
CHUNK = 128
N_CHUNKS = E_PER_TEC // CHUNK
N_PAIRS = N_CHUNKS // 2
ROWS_PER_TEC = N_PAD // N_TEC
E_PER_W = E_PAD // N_WORKER
LANES = 16
PHASES = 2
CPP = N_CHUNKS // PHASES


@functools.partial(
    pl.kernel,
    mesh=_SC_MESH,
    out_type=jax.ShapeDtypeStruct((N_WORKER, N_PAD), jnp.float32),
    scratch_types=[
        pltpu.VMEM((E_PER_W,), jnp.int32),
        pltpu.VMEM((N_PAD,), jnp.float32),
    ],
    compiler_params=_SC_CP,
)
def _sc_degree(dst_hbm, out_hbm, idx_v, hist_v):
    c = lax.axis_index("c")
    s = lax.axis_index("s")
    wid = s * 2 + c
    zeros = jnp.zeros((LANES,), jnp.float32)
    ones = jnp.full((LANES,), 1.0, jnp.float32)

    @pl.loop(0, N_PAD // LANES)
    def _(i):
        hist_v[pl.ds(i * LANES, LANES)] = zeros

    pltpu.sync_copy(dst_hbm.at[pl.ds(wid * E_PER_W, E_PER_W)], idx_v)

    @pl.loop(0, E_PER_W // LANES)
    def _(i):
        idx = idx_v[pl.ds(i * LANES, LANES)]
        plsc.addupdate_scatter(hist_v, [idx], ones)

    pltpu.sync_copy(hist_v, out_hbm.at[wid])


def _make_sc_scatter(with_gather):
    if with_gather:
        out_type = jax.ShapeDtypeStruct((2, B_PAD, DH), jnp.float32)
    else:
        out_type = jax.ShapeDtypeStruct((2 * N_PAD, DH), jnp.float32)

    @functools.partial(
        pl.kernel,
        mesh=_SC_MESH,
        out_type=out_type,
        scratch_types=[
            pltpu.VMEM((CPP, CHUNK), jnp.int32),
            pltpu.VMEM((CPP, CHUNK), jnp.int32),
            pltpu.VMEM((CHUNK, DH), jnp.float32),
            pltpu.VMEM((CHUNK, DH), jnp.float32),
            pltpu.VMEM_SHARED((N_PAD, DH), jnp.float32),
            pltpu.SemaphoreType.DMA,
            pltpu.SemaphoreType.DMA,
        ],
        compiler_params=_SC_CP,
    )
    def _sc_scatter(p_hbm, srcp_hbm, dstp_hbm, fetch_hbm, out_hbm,
                    src_v, dst_v, rows0, rows1, acc_sh, sr0, sr1):
        c = lax.axis_index("c")
        s = lax.axis_index("s")
        zeros = jnp.zeros((LANES,), jnp.float32)

        @pl.loop(0, CHUNK)
        def _(r):
            @pl.loop(0, DH // LANES)
            def _(k):
                rows0[r, pl.ds(k * LANES, LANES)] = zeros

        @pl.loop(0, ROWS_PER_TEC // CHUNK)
        def _(b):
            pltpu.sync_copy(
                rows0,
                acc_sh.at[pl.ds(s * ROWS_PER_TEC + b * CHUNK, CHUNK)])

        plsc.subcore_barrier()

        def gat(j, rows, sem):
            return pltpu.make_async_copy(p_hbm.at[src_v.at[j]], rows, sem)

        def sca(j, rows):
            pltpu.sync_copy(rows, acc_sh.at[dst_v.at[j]], add=True)

        for p in range(PHASES):
            pltpu.sync_copy(srcp_hbm.at[c, s, p], src_v)
            pltpu.sync_copy(dstp_hbm.at[s, p], dst_v)
            gat(0, rows0, sr0).start()

            @pl.loop(0, CPP // 2 - 1)
            def _(k):
                j0 = 2 * k
                gat(j0 + 1, rows1, sr1).start()
                gat(j0, rows0, sr0).wait()
                sca(j0, rows0)
                gat(j0 + 2, rows0, sr0).start()
                gat(j0 + 1, rows1, sr1).wait()
                sca(j0 + 1, rows1)

            gat(CPP - 1, rows1, sr1).start()
            gat(CPP - 2, rows0, sr0).wait()
            sca(CPP - 2, rows0)
            gat(CPP - 1, rows1, sr1).wait()
            sca(CPP - 1, rows1)

        plsc.subcore_barrier()
        if with_gather:
            @pl.when(s == 0)
            def _():
                pltpu.sync_copy(fetch_hbm, src_v.at[0])
                pltpu.async_copy(acc_sh.at[src_v.at[0]], rows0, sr0).wait()
                pltpu.sync_copy(rows0, out_hbm.at[c])
        else:
            pltpu.sync_copy(
                acc_sh.at[pl.ds(s * ROWS_PER_TEC, ROWS_PER_TEC)],
                out_hbm.at[pl.ds(c * N_PAD + s * ROWS_PER_TEC, ROWS_PER_TEC)])

    return _sc_scatter


_sc_scatter_plain = _make_sc_scatter(False)
_sc_scatter_gath = _make_sc_scatter(True)


def _deg_parts(dst_pad_flat):
    return _sc_degree(dst_pad_flat)


def _scatter_parts(p_parts, srcp, dstp, fetch_idx):
    p_flat = p_parts.reshape(2 * N_PAD, DH)
    s_flat = _sc_scatter_plain(p_flat, srcp, dstp, fetch_idx)
    return s_flat.reshape(2, N_PAD, DH)


def _scatter_parts_gath(p_parts, srcp, dstp, fetch_idx):
    p_flat = p_parts.reshape(2 * N_PAD, DH)
    return _sc_scatter_gath(p_flat, srcp, dstp, fetch_idx)


def kernel(features, edge_index, to_fetch, W1, b1, W2, b2, Wl, bl):
    src = edge_index[0].astype(jnp.int32)
    dst = edge_index[1].astype(jnp.int32)

    junk = N_RAW + jnp.arange(E_PAD - E_RAW, dtype=jnp.int32) % (N_PAD - N_RAW)
    src_pad = jnp.concatenate([src, junk])
    dst_pad = jnp.concatenate([dst, junk])
    srcp = jnp.stack([src_pad, src_pad + N_PAD]).reshape(
        2, N_TEC, PHASES, CPP, CHUNK)
    dstp = dst_pad.reshape(N_TEC, PHASES, CPP, CHUNK)

    feats = jnp.zeros((N_PAD, D), jnp.float32).at[:N_RAW].set(
        features.astype(jnp.float32))
    fetch_idx = jnp.zeros((B_PAD,), jnp.int32).at[:B_G].set(
        to_fetch.astype(jnp.int32)
        + jnp.arange(B_G, dtype=jnp.int32) * (N_RAW // B_G))

    deg_parts = _deg_parts(dst_pad)

    m1 = _tc_matmul(feats, W1)
    p1, norm = _tc_scale_split(m1, deg_parts)
    s1 = _scatter_parts(p1, srcp, dstp, fetch_idx)
    p2 = _tc_layer2(s1, norm, b1.reshape(1, D), W2)
    gath2 = _scatter_parts_gath(p2, srcp, dstp, fetch_idx)
    out = _tc_readout(fetch_idx, gath2, norm, b2.reshape(1, D),
                      Wl.T, bl.reshape(1, D_OUT))
    h = out[:B_G]
    return (h, h)


# speedup vs baseline: 1.0560x; 1.0083x over previous
import dataclasses
import functools

import jax
import jax.numpy as jnp
from jax import lax
from jax.experimental import pallas as pl
from jax.experimental.pallas import tpu as pltpu
from jax.experimental.pallas import tpu_sc as plsc

N_RAW = 10000
N_PAD = 10240
E_RAW = 160000
E_PAD = 163840
D = 256
DH = 128
D_OUT = 64
B_G = 100
B_PAD = 128
ROW_BLK = 512
N_BLKS = N_PAD // ROW_BLK


def _mm_body(x_ref, w_ref, o_ref):
    o_ref[...] = jnp.dot(x_ref[...], w_ref[...],
                         preferred_element_type=jnp.float32)


def _tc_matmul(x, w):
    return pl.pallas_call(
        _mm_body,
        grid=(N_BLKS,),
        in_specs=[
            pl.BlockSpec((ROW_BLK, D), lambda i: (i, 0)),
            pl.BlockSpec((D, D), lambda i: (0, 0)),
        ],
        out_specs=pl.BlockSpec((ROW_BLK, D), lambda i: (i, 0)),
        out_shape=jax.ShapeDtypeStruct((N_PAD, D), jnp.float32),
    )(x, w)


def _scale_body(m_ref, deg_ref, p_ref, norm_ref):
    deg = jnp.sum(deg_ref[...], axis=0)
    norm = lax.rsqrt(jnp.maximum(deg, 1.0))[:, None]
    p = m_ref[...] * norm
    p_ref[0] = p[:, :DH]
    p_ref[1] = p[:, DH:]
    norm_ref[...] = norm


def _tc_scale_split(m, deg_parts):
    return pl.pallas_call(
        _scale_body,
        grid=(N_BLKS,),
        in_specs=[
            pl.BlockSpec((ROW_BLK, D), lambda i: (i, 0)),
            pl.BlockSpec((N_WORKER, ROW_BLK), lambda i: (0, i)),
        ],
        out_specs=[
            pl.BlockSpec((2, ROW_BLK, DH), lambda i: (0, i, 0)),
            pl.BlockSpec((ROW_BLK, 1), lambda i: (i, 0)),
        ],
        out_shape=[
            jax.ShapeDtypeStruct((2, N_PAD, DH), jnp.float32),
            jax.ShapeDtypeStruct((N_PAD, 1), jnp.float32),
        ],
    )(m, deg_parts)


def _layer2_body(s_ref, norm_ref, b_ref, w_ref, p_ref):
    norm = norm_ref[...]
    x = jnp.concatenate([s_ref[0], s_ref[1]], axis=1)
    h = jax.nn.relu(x * norm + b_ref[...]) * norm
    p = jnp.dot(h, w_ref[...], preferred_element_type=jnp.float32)
    p_ref[0] = p[:, :DH]
    p_ref[1] = p[:, DH:]


def _tc_layer2(s_parts, norm, b1, w2):
    return pl.pallas_call(
        _layer2_body,
        grid=(N_BLKS,),
        in_specs=[
            pl.BlockSpec((2, ROW_BLK, DH), lambda i: (0, i, 0)),
            pl.BlockSpec((ROW_BLK, 1), lambda i: (i, 0)),
            pl.BlockSpec((1, D), lambda i: (0, 0)),
            pl.BlockSpec((D, D), lambda i: (0, 0)),
        ],
        out_specs=pl.BlockSpec((2, ROW_BLK, DH), lambda i: (0, i, 0)),
        out_shape=jax.ShapeDtypeStruct((2, N_PAD, DH), jnp.float32),
    )(s_parts, norm, b1, w2)


def _readout_body(fetch_ref, g_ref, norm_ref, b_ref, wlt_ref, bl_ref,
                  o_ref, gn):
    def body(i, _):
        idx = fetch_ref[i]
        gn[pl.ds(i, 1), :] = norm_ref[pl.ds(idx, 1), :]
        return 0

    lax.fori_loop(0, B_PAD, body, 0)
    x = jnp.concatenate([g_ref[0], g_ref[1]], axis=1)
    h = jax.nn.relu(x * gn[...] + b_ref[...])
    o_ref[...] = jnp.dot(h, wlt_ref[...],
                         preferred_element_type=jnp.float32) + bl_ref[...]


def _tc_readout(fetch_idx, gath, norm, b2, wl_t, bl):
    return pl.pallas_call(
        _readout_body,
        in_specs=[
            pl.BlockSpec(memory_space=pltpu.SMEM),
            pl.BlockSpec((2, B_PAD, DH), lambda: (0, 0, 0)),
            pl.BlockSpec((N_PAD, 1), lambda: (0, 0)),
            pl.BlockSpec((1, D), lambda: (0, 0)),
            pl.BlockSpec((D, D_OUT), lambda: (0, 0)),
            pl.BlockSpec((1, D_OUT), lambda: (0, 0)),
        ],
        out_specs=pl.BlockSpec((B_PAD, D_OUT), lambda: (0, 0)),
        out_shape=jax.ShapeDtypeStruct((B_PAD, D_OUT), jnp.float32),
        scratch_shapes=[
            pltpu.VMEM((B_PAD, 1), jnp.float32),
        ],
    )(fetch_idx, gath, norm, b2, wl_t, bl)


_SC_MESH = plsc.VectorSubcoreMesh(core_axis_name="c", subcore_axis_name="s")
_SC_CP = pltpu.CompilerParams()
if "needs_layout_passes" in pltpu.CompilerParams.__dataclass_fields__:
    _SC_CP = dataclasses.replace(_SC_CP, needs_layout_passes=False)
N_TEC = 16
N_WORKER = 32
E_PER_TEC = E_PAD // N_TEC
CHUNK = 128
N_CHUNKS = E_PER_TEC // CHUNK
N_PAIRS = N_CHUNKS // 2
ROWS_PER_TEC = N_PAD // N_TEC
E_PER_W = E_PAD // N_WORKER
LANES = 16
PHASES = 2
CPP = N_CHUNKS // PHASES


@functools.partial(
    pl.kernel,
    mesh=_SC_MESH,
    out_type=jax.ShapeDtypeStruct((N_WORKER, N_PAD), jnp.float32),
    scratch_types=[
        pltpu.VMEM((E_PER_W,), jnp.int32),
        pltpu.VMEM((N_PAD,), jnp.float32),
    ],
    compiler_params=_SC_CP,
)
def _sc_degree(dst_hbm, out_hbm, idx_v, hist_v):
    c = lax.axis_index("c")
    s = lax.axis_index("s")
    wid = s * 2 + c
    zeros = jnp.zeros((LANES,), jnp.float32)
    ones = jnp.full((LANES,), 1.0, jnp.float32)

    @pl.loop(0, N_PAD // LANES)
    def _(i):
        hist_v[pl.ds(i * LANES, LANES)] = zeros

    pltpu.sync_copy(dst_hbm.at[pl.ds(wid * E_PER_W, E_PER_W)], idx_v)

    @pl.loop(0, E_PER_W // LANES)
    def _(i):
        idx = idx_v[pl.ds(i * LANES, LANES)]
        plsc.addupdate_scatter(hist_v, [idx], ones)

    pltpu.sync_copy(hist_v, out_hbm.at[wid])


def _make_sc_scatter(with_gather):
    if with_gather:
        out_type = jax.ShapeDtypeStruct((2, B_PAD, DH), jnp.float32)
    else:
        out_type = jax.ShapeDtypeStruct((2 * N_PAD, DH), jnp.float32)

    @functools.partial(
        pl.kernel,
        mesh=_SC_MESH,
        out_type=out_type,
        scratch_types=[
            pltpu.VMEM((CPP, CHUNK), jnp.int32),
            pltpu.VMEM((CPP, CHUNK), jnp.int32),
            pltpu.VMEM((CHUNK, DH), jnp.float32),
            pltpu.VMEM((CHUNK, DH), jnp.float32),
            pltpu.VMEM_SHARED((N_PAD, DH), jnp.float32),
            pltpu.SemaphoreType.DMA,
            pltpu.SemaphoreType.DMA,
        ],
        compiler_params=_SC_CP,
    )
    def _sc_scatter(p_hbm, srcp_hbm, dstp_hbm, fetch_hbm, out_hbm,
                    src_v, dst_v, rows0, rows1, acc_sh, sr0, sr1):
        c = lax.axis_index("c")
        s = lax.axis_index("s")
        zeros = jnp.zeros((LANES,), jnp.float32)

        @pl.loop(0, CHUNK)
        def _(r):
            @pl.loop(0, DH // LANES)
            def _(k):
                rows0[r, pl.ds(k * LANES, LANES)] = zeros

        pltpu.async_copy(srcp_hbm.at[c, s, 0], src_v, sr1)
        for b in range(ROWS_PER_TEC // CHUNK):
            pltpu.async_copy(
                rows0,
                acc_sh.at[pl.ds(s * ROWS_PER_TEC + b * CHUNK, CHUNK)], sr0)
        pltpu.sync_copy(dstp_hbm.at[s, 0], dst_v)
        pltpu.make_async_copy(srcp_hbm.at[c, s, 0], src_v, sr1).wait()
        for b in range(ROWS_PER_TEC // CHUNK):
            pltpu.make_async_copy(
                rows0,
                acc_sh.at[pl.ds(s * ROWS_PER_TEC + b * CHUNK, CHUNK)],
                sr0).wait()

        plsc.subcore_barrier()

        def gat(j, rows, sem):
            return pltpu.make_async_copy(p_hbm.at[src_v.at[j]], rows, sem)

        def sca(j, rows):
            pltpu.sync_copy(rows, acc_sh.at[dst_v.at[j]], add=True)

        for p in range(PHASES):
            if p > 0:
                pltpu.sync_copy(srcp_hbm.at[c, s, p], src_v)
                pltpu.sync_copy(dstp_hbm.at[s, p], dst_v)
            gat(0, rows0, sr0).start()

            @pl.loop(0, CPP // 2 - 1)
            def _(k):
                j0 = 2 * k
                gat(j0 + 1, rows1, sr1).start()
                gat(j0, rows0, sr0).wait()
                sca(j0, rows0)
                gat(j0 + 2, rows0, sr0).start()
                gat(j0 + 1, rows1, sr1).wait()
                sca(j0 + 1, rows1)

            gat(CPP - 1, rows1, sr1).start()
            gat(CPP - 2, rows0, sr0).wait()
            sca(CPP - 2, rows0)
            gat(CPP - 1, rows1, sr1).wait()
            sca(CPP - 1, rows1)

        plsc.subcore_barrier()
        if with_gather:
            @pl.when(s == 0)
            def _():
                pltpu.sync_copy(fetch_hbm, src_v.at[0])
                pltpu.async_copy(acc_sh.at[src_v.at[0]], rows0, sr0).wait()
                pltpu.sync_copy(rows0, out_hbm.at[c])
        else:
            pltpu.sync_copy(
                acc_sh.at[pl.ds(s * ROWS_PER_TEC, ROWS_PER_TEC)],
                out_hbm.at[pl.ds(c * N_PAD + s * ROWS_PER_TEC, ROWS_PER_TEC)])

    return _sc_scatter


_sc_scatter_plain = _make_sc_scatter(False)
_sc_scatter_gath = _make_sc_scatter(True)


def _deg_parts(dst_pad_flat):
    return _sc_degree(dst_pad_flat)


def _scatter_parts(p_parts, srcp, dstp, fetch_idx):
    p_flat = p_parts.reshape(2 * N_PAD, DH)
    s_flat = _sc_scatter_plain(p_flat, srcp, dstp, fetch_idx)
    return s_flat.reshape(2, N_PAD, DH)


def _scatter_parts_gath(p_parts, srcp, dstp, fetch_idx):
    p_flat = p_parts.reshape(2 * N_PAD, DH)
    return _sc_scatter_gath(p_flat, srcp, dstp, fetch_idx)


def kernel(features, edge_index, to_fetch, W1, b1, W2, b2, Wl, bl):
    src = edge_index[0].astype(jnp.int32)
    dst = edge_index[1].astype(jnp.int32)

    junk = N_RAW + jnp.arange(E_PAD - E_RAW, dtype=jnp.int32) % (N_PAD - N_RAW)
    src_pad = jnp.concatenate([src, junk])
    dst_pad = jnp.concatenate([dst, junk])
    srcp = jnp.stack([src_pad, src_pad + N_PAD]).reshape(
        2, N_TEC, PHASES, CPP, CHUNK)
    dstp = dst_pad.reshape(N_TEC, PHASES, CPP, CHUNK)

    feats = jnp.zeros((N_PAD, D), jnp.float32).at[:N_RAW].set(
        features.astype(jnp.float32))
    fetch_idx = jnp.zeros((B_PAD,), jnp.int32).at[:B_G].set(
        to_fetch.astype(jnp.int32)
        + jnp.arange(B_G, dtype=jnp.int32) * (N_RAW // B_G))

    deg_parts = _deg_parts(dst_pad)

    m1 = _tc_matmul(feats, W1)
    p1, norm = _tc_scale_split(m1, deg_parts)
    s1 = _scatter_parts(p1, srcp, dstp, fetch_idx)
    p2 = _tc_layer2(s1, norm, b1.reshape(1, D), W2)
    gath2 = _scatter_parts_gath(p2, srcp, dstp, fetch_idx)
    out = _tc_readout(fetch_idx, gath2, norm, b2.reshape(1, D),
                      Wl.T, bl.reshape(1, D_OUT))
    h = out[:B_G]
    return (h, h)
